# baseline (speedup 1.0000x reference)
import jax
import jax.numpy as jnp
from jax import lax
from jax.experimental import pallas as pl
from jax.experimental.pallas import tpu as pltpu
from jax.experimental.pallas import tpu_sc as plsc

_K = 64
_M = 128
_N = 32768

_NC, _NS, _L = 2, 16, 16
_NW = _NC * _NS
_RPW = _M // _NW
_NB = 2048
_HV = _NB // _L
_SV = _HV // _L
_CAP = 4096
_LOWB = 21
_IMIN = jnp.int32(-2147483648)

_BR = 8


def _sortable(v):
    u = lax.bitcast_convert_type(v, jnp.uint32)
    neg = (u >> jnp.uint32(31)) > jnp.uint32(0)
    return jnp.where(neg, ~u, u | jnp.uint32(0x80000000))


def _flip(su):
    return lax.bitcast_convert_type(su ^ jnp.uint32(0x80000000), jnp.int32)


def _unsortable(su):
    pos = (su >> jnp.uint32(31)) > jnp.uint32(0)
    u = jnp.where(pos, su & jnp.uint32(0x7FFFFFFF), ~su)
    return lax.bitcast_convert_type(u, jnp.float32)


def _sc_body(x_hbm, thr_hbm, row_a, row_b, hist_v, sums_v, su_v, thr_v,
             sin_a, sin_b, sout):
    c = lax.axis_index("c")
    s = lax.axis_index("s")
    wid = s * _NC + c
    base = wid * _RPW
    lanes = lax.iota(jnp.int32, _L)
    bufs = (row_a, row_b)
    sins = (sin_a, sin_b)

    def hist_pass(buf):
        @plsc.parallel_loop(0, _NB, step=_L, unroll=8)
        def _(i):
            hist_v[pl.ds(i, _L)] = jnp.zeros((_L,), jnp.int32)

        ones = jnp.ones((_L,), jnp.int32)

        @plsc.parallel_loop(0, _N, step=_L, unroll=8)
        def _(i):
            su = _sortable(buf[pl.ds(i, _L)])
            b = (su >> jnp.uint32(_LOWB)).astype(jnp.int32)
            plsc.addupdate_scatter(hist_v, [b], ones)

    def find_bucket(t_lvl, r):
        @plsc.parallel_loop(0, _SV, unroll=1)
        def _(i):
            sums_v[pl.ds(i * _L, _L)] = jnp.zeros((_L,), jnp.int32)

        lane0 = lanes == 0

        @plsc.parallel_loop(0, _HV, unroll=4)
        def _(i):
            hv = hist_v[pl.ds(i * _L, _L)]
            sv = jnp.full((_L,), jnp.sum(hv, axis=0))
            iv = jnp.full((_L,), i, jnp.int32)
            plsc.addupdate_scatter(sums_v, [iv], sv, mask=lane0)

        lim = t_lvl - r
        pre = jnp.int32(0)
        pres = []
        nhits = jnp.int32(0)
        for i in range(_SV):
            sv = sums_v[pl.ds(i * _L, _L)]
            cs = plsc.cumsum(sv)
            pres.append(pre)
            hit = (pre + cs - sv) <= lim
            nhits = nhits + plsc.all_reduce_population_count(hit)[0]
            pre = pre + cs[_L - 1]
        gidx = nhits - 1
        gv = gidx // _L
        gl = gidx % _L
        pre_g = jnp.int32(0)
        for i in range(_SV):
            pre_g = jnp.where(gv == i, pres[i], pre_g)
        sv = sums_v[pl.ds(gv * _L, _L)]
        cs = plsc.cumsum(sv)
        excl = pre_g + cs - sv
        pre_grp = jnp.sum(jnp.where(lanes == gl, excl, 0), axis=0)

        hv = hist_v[pl.ds(gidx * _L, _L)]
        hcs = plsc.cumsum(hv)
        hexcl = pre_grp + hcs - hv
        hhit = hexcl <= lim
        lsel = plsc.all_reduce_population_count(hhit)[0] - 1
        return gidx * _L + lsel

    def compact(buf, b1):
        b1u = b1.astype(jnp.uint32)
        capv = jnp.full((_L,), _CAP, jnp.int32)

        @plsc.parallel_loop(0, _N, step=_L, unroll=4,
                            carry=jnp.zeros((_L,), jnp.int32))
        def off(i, off):
            su = _sortable(buf[pl.ds(i, _L)])
            m = (su >> jnp.uint32(_LOWB)) >= b1u
            mi = m.astype(jnp.int32)
            excl = plsc.cumsum(mi) - mi
            idx = jnp.minimum(off + excl, capv)
            plsc.store_scatter(su_v, [idx], _flip(su), mask=m)
            return off + plsc.all_reduce_population_count(m)

        ncand = jnp.sum(jnp.where(lanes == 0, off, 0), axis=0)
        safe = jnp.minimum(ncand, jnp.int32(_CAP))
        su_v[pl.ds(safe, _L)] = jnp.full((_L,), _IMIN)
        return ncand

    def refine_cand(nv, b1u):
        def outer(b, v):
            cand_low = v | (jnp.int32(1) << (jnp.int32(_LOWB - 1) - b))
            tsu = (b1u << jnp.uint32(_LOWB)) | cand_low.astype(jnp.uint32)
            tf = jnp.full((_L,), _flip(tsu))

            def inner(j, acc):
                sf = su_v[pl.ds(j * _L, _L)]
                m = sf >= tf
                return acc + plsc.all_reduce_population_count(m)[0]

            cnt = lax.fori_loop(0, nv, inner, jnp.int32(0))
            return jnp.where(cnt >= jnp.int32(_K), cand_low, v)

        return lax.fori_loop(0, _LOWB, outer, jnp.int32(0))

    def refine_row(buf, b1u):
        hi = b1u << jnp.uint32(_LOWB)

        def outer(b, v):
            cand_low = v | (jnp.int32(1) << (jnp.int32(_LOWB - 1) - b))
            tfull = hi | cand_low.astype(jnp.uint32)

            def inner(j, acc):
                su = _sortable(buf[pl.ds(j * _L, _L)])
                m = su >= tfull
                return acc + plsc.all_reduce_population_count(m)[0]

            cnt = lax.fori_loop(0, _N // _L, inner, jnp.int32(0))
            return jnp.where(cnt >= jnp.int32(_K), cand_low, v)

        return lax.fori_loop(0, _LOWB, outer, jnp.int32(0))

    in_copies = [None] * _RPW
    in_copies[0] = pltpu.async_copy(x_hbm.at[base], bufs[0], sins[0])
    in_copies[1] = pltpu.async_copy(x_hbm.at[base + 1], bufs[1], sins[1])
    for k in range(_RPW):
        buf = bufs[k % 2]
        in_copies[k].wait()
        hist_pass(buf)
        b1 = find_bucket(jnp.int32(_N), jnp.int32(_K))
        b1u = b1.astype(jnp.uint32)
        ncand = compact(buf, b1)
        is_fb = ncand > jnp.int32(_CAP)
        nv = (jnp.minimum(ncand, jnp.int32(_CAP)) + _L - 1) // _L
        vlow = lax.cond(
            is_fb,
            lambda: refine_row(buf, b1u),
            lambda: refine_cand(nv, b1u),
        )
        tsu = (b1u << jnp.uint32(_LOWB)) | vlow.astype(jnp.uint32)
        thr_v[pl.ds(k * _L, _L)] = jnp.full((_L,), _unsortable(tsu))
        if k + 2 < _RPW:
            in_copies[k + 2] = pltpu.async_copy(
                x_hbm.at[base + k + 2], bufs[k % 2], sins[k % 2])
    out_copy = pltpu.async_copy(thr_v, thr_hbm.at[wid], sout)
    out_copy.wait()


def _mask_body(x_ref, t_ref, o_ref):
    o_ref[...] = jnp.where(x_ref[...] >= t_ref[...], x_ref[...], 0.0)


@jax.jit
def kernel(x):
    m, n = x.shape
    sc_run = pl.kernel(
        _sc_body,
        out_type=jax.ShapeDtypeStruct((_NW, _RPW * _L), jnp.float32),
        mesh=plsc.VectorSubcoreMesh(core_axis_name="c", subcore_axis_name="s"),
        compiler_params=pltpu.CompilerParams(needs_layout_passes=False),
        scratch_types=[
            pltpu.VMEM((_N,), jnp.float32),
            pltpu.VMEM((_N,), jnp.float32),
            pltpu.VMEM((_NB,), jnp.int32),
            pltpu.VMEM((_SV * _L,), jnp.int32),
            pltpu.VMEM((_CAP + 2 * _L,), jnp.int32),
            pltpu.VMEM((_RPW * _L,), jnp.float32),
            pltpu.SemaphoreType.DMA,
            pltpu.SemaphoreType.DMA,
            pltpu.SemaphoreType.DMA,
        ],
    )
    thrs = sc_run(x)
    thr = thrs.reshape(_NW, _RPW, _L)[:, :, 0].reshape(m, 1)
    out = pl.pallas_call(
        _mask_body,
        grid=(m // _BR,),
        in_specs=[
            pl.BlockSpec((_BR, n), lambda i: (i, 0)),
            pl.BlockSpec((_BR, 1), lambda i: (i, 0)),
        ],
        out_specs=pl.BlockSpec((_BR, n), lambda i: (i, 0)),
        out_shape=jax.ShapeDtypeStruct((m, n), jnp.float32),
        compiler_params=pltpu.CompilerParams(
            dimension_semantics=("arbitrary",),
        ),
    )(x, thr)
    return out


# speedup vs baseline: 1.1995x; 1.1995x over previous
import jax
import jax.numpy as jnp
from jax import lax
from jax.experimental import pallas as pl
from jax.experimental.pallas import tpu as pltpu
from jax.experimental.pallas import tpu_sc as plsc

_K = 64
_M = 128
_N = 32768

_NC, _NS, _L = 2, 16, 16
_NW = _NC * _NS
_RPW = _M // _NW
_NB = 2048
_HV = _NB // _L
_SV = _HV // _L
_CAP = 4096
_LOWB = 21
_IMIN = jnp.int32(-2147483648)
_NCH = 4
_CH = _N // _NCH


def _sortable(v):
    u = lax.bitcast_convert_type(v, jnp.uint32)
    neg = (u >> jnp.uint32(31)) > jnp.uint32(0)
    return jnp.where(neg, ~u, u | jnp.uint32(0x80000000))


def _flip(su):
    return lax.bitcast_convert_type(su ^ jnp.uint32(0x80000000), jnp.int32)


def _unsortable(su):
    pos = (su >> jnp.uint32(31)) > jnp.uint32(0)
    u = jnp.where(pos, su & jnp.uint32(0x7FFFFFFF), ~su)
    return lax.bitcast_convert_type(u, jnp.float32)


def _sc_body(x_hbm, out_hbm, row_a, row_b, out_v, hist_v, sums_v, su_v,
             pos_a, pos_b, sin_a, sin_b, sout):
    c = lax.axis_index("c")
    s = lax.axis_index("s")
    wid = s * _NC + c
    base = wid * _RPW
    lanes = lax.iota(jnp.int32, _L)
    bufs = (row_a, row_b)
    sins = (sin_a, sin_b)
    poss = (pos_a, pos_b)
    fzeros = jnp.zeros((_L,), jnp.float32)

    @plsc.parallel_loop(0, _N, step=_L, unroll=8)
    def _(i):
        out_v[pl.ds(i, _L)] = fzeros

    def hist_pass(buf):
        @plsc.parallel_loop(0, _NB, step=_L, unroll=8)
        def _(i):
            hist_v[pl.ds(i, _L)] = jnp.zeros((_L,), jnp.int32)

        ones = jnp.ones((_L,), jnp.int32)

        @plsc.parallel_loop(0, _N, step=_L, unroll=8)
        def _(i):
            su = _sortable(buf[pl.ds(i, _L)])
            b = (su >> jnp.uint32(_LOWB)).astype(jnp.int32)
            plsc.addupdate_scatter(hist_v, [b], ones)

    def find_bucket(t_lvl, r):
        @plsc.parallel_loop(0, _SV, unroll=1)
        def _(i):
            sums_v[pl.ds(i * _L, _L)] = jnp.zeros((_L,), jnp.int32)

        lane0 = lanes == 0

        @plsc.parallel_loop(0, _HV, unroll=4)
        def _(i):
            hv = hist_v[pl.ds(i * _L, _L)]
            sv = jnp.full((_L,), jnp.sum(hv, axis=0))
            iv = jnp.full((_L,), i, jnp.int32)
            plsc.addupdate_scatter(sums_v, [iv], sv, mask=lane0)

        lim = t_lvl - r
        pre = jnp.int32(0)
        pres = []
        nhits = jnp.int32(0)
        for i in range(_SV):
            sv = sums_v[pl.ds(i * _L, _L)]
            cs = plsc.cumsum(sv)
            pres.append(pre)
            hit = (pre + cs - sv) <= lim
            nhits = nhits + plsc.all_reduce_population_count(hit)[0]
            pre = pre + cs[_L - 1]
        gidx = nhits - 1
        gv = gidx // _L
        gl = gidx % _L
        pre_g = jnp.int32(0)
        for i in range(_SV):
            pre_g = jnp.where(gv == i, pres[i], pre_g)
        sv = sums_v[pl.ds(gv * _L, _L)]
        cs = plsc.cumsum(sv)
        excl = pre_g + cs - sv
        pre_grp = jnp.sum(jnp.where(lanes == gl, excl, 0), axis=0)

        hv = hist_v[pl.ds(gidx * _L, _L)]
        hcs = plsc.cumsum(hv)
        hexcl = pre_grp + hcs - hv
        hhit = hexcl <= lim
        lsel = plsc.all_reduce_population_count(hhit)[0] - 1
        return gidx * _L + lsel

    def compact(buf, b1, pos_v):
        b1u = b1.astype(jnp.uint32)
        capv = jnp.full((_L,), _CAP, jnp.int32)

        @plsc.parallel_loop(0, _N, step=_L, unroll=4,
                            carry=jnp.zeros((_L,), jnp.int32))
        def off(i, off):
            su = _sortable(buf[pl.ds(i, _L)])
            m = (su >> jnp.uint32(_LOWB)) >= b1u
            mi = m.astype(jnp.int32)
            excl = plsc.cumsum(mi) - mi
            idx = jnp.minimum(off + excl, capv)
            plsc.store_scatter(su_v, [idx], _flip(su), mask=m)
            plsc.store_scatter(pos_v, [idx], i + lanes, mask=m)
            return off + plsc.all_reduce_population_count(m)

        ncand = jnp.sum(jnp.where(lanes == 0, off, 0), axis=0)
        safe = jnp.minimum(ncand, jnp.int32(_CAP))
        su_v[pl.ds(safe, _L)] = jnp.full((_L,), _IMIN)
        pos_v[pl.ds(safe, _L)] = jnp.zeros((_L,), jnp.int32)
        return ncand

    def refine_cand(nv, b1u):
        def outer(b, v):
            cand_low = v | (jnp.int32(1) << (jnp.int32(_LOWB - 1) - b))
            tsu = (b1u << jnp.uint32(_LOWB)) | cand_low.astype(jnp.uint32)
            tf = jnp.full((_L,), _flip(tsu))

            def inner(j, acc):
                sf = su_v[pl.ds(j * _L, _L)]
                m = sf >= tf
                return acc + plsc.all_reduce_population_count(m)[0]

            cnt = lax.fori_loop(0, nv, inner, jnp.int32(0))
            return jnp.where(cnt >= jnp.int32(_K), cand_low, v)

        return lax.fori_loop(0, _LOWB, outer, jnp.int32(0))

    def refine_row(buf, b1u):
        hi = b1u << jnp.uint32(_LOWB)

        def outer(b, v):
            cand_low = v | (jnp.int32(1) << (jnp.int32(_LOWB - 1) - b))
            tfull = hi | cand_low.astype(jnp.uint32)

            def inner(j, acc):
                su = _sortable(buf[pl.ds(j * _L, _L)])
                m = su >= tfull
                return acc + plsc.all_reduce_population_count(m)[0]

            cnt = lax.fori_loop(0, _N // _L, inner, jnp.int32(0))
            return jnp.where(cnt >= jnp.int32(_K), cand_low, v)

        return lax.fori_loop(0, _LOWB, outer, jnp.int32(0))

    def load_row(k):
        buf = bufs[k % 2]
        sem = sins[k % 2]
        return [
            pltpu.async_copy(
                x_hbm.at[base + k, pl.ds(ci * _CH, _CH)],
                buf.at[pl.ds(ci * _CH, _CH)], sem.at[ci])
            for ci in range(_NCH)
        ]

    in_copies = [None] * _RPW
    out_copy = None
    prev = None
    in_copies[0] = load_row(0)
    in_copies[1] = load_row(1)
    for k in range(_RPW):
        buf = bufs[k % 2]
        pos_v = poss[k % 2]
        for cp in in_copies[k]:
            cp.wait()
        hist_pass(buf)
        b1 = find_bucket(jnp.int32(_N), jnp.int32(_K))
        b1u = b1.astype(jnp.uint32)
        ncand = compact(buf, b1, pos_v)
        is_fb = ncand > jnp.int32(_CAP)
        ncand_c = jnp.minimum(ncand, jnp.int32(_CAP))
        nv = (ncand_c + _L - 1) // _L
        vlow = lax.cond(
            is_fb,
            lambda: refine_row(buf, b1u),
            lambda: refine_cand(nv, b1u),
        )
        tsu = (b1u << jnp.uint32(_LOWB)) | vlow.astype(jnp.uint32)
        thr_fv = jnp.full((_L,), _flip(tsu))
        thr_vec = jnp.full((_L,), _unsortable(tsu))

        if k > 0:
            out_copy.wait()
            p_fb, p_pos, p_ncand, p_nv = prev

            def rz_scatter(p_pos=p_pos, p_ncand=p_ncand, p_nv=p_nv):
                def body(j, _):
                    pv = p_pos[pl.ds(j * _L, _L)]
                    m = (j * _L + lanes) < p_ncand
                    plsc.store_scatter(out_v, [pv], fzeros, mask=m)
                    return jnp.int32(0)

                return lax.fori_loop(0, p_nv, body, jnp.int32(0))

            def rz_full():
                def body(j, _):
                    out_v[pl.ds(j * _L, _L)] = fzeros
                    return jnp.int32(0)

                return lax.fori_loop(0, _N // _L, body, jnp.int32(0))

            lax.cond(p_fb, rz_full, rz_scatter)

        def wr_scatter(nv=nv, ncand=ncand_c, pos_v=pos_v, thr_fv=thr_fv):
            def body(j, _):
                sf = su_v[pl.ds(j * _L, _L)]
                pv = pos_v[pl.ds(j * _L, _L)]
                su = lax.bitcast_convert_type(sf, jnp.uint32)
                val = _unsortable(su ^ jnp.uint32(0x80000000))
                m = (sf >= thr_fv) & ((j * _L + lanes) < ncand)
                plsc.store_scatter(out_v, [pv], val, mask=m)
                return jnp.int32(0)

            return lax.fori_loop(0, nv, body, jnp.int32(0))

        def wr_full(buf=buf, thr_vec=thr_vec):
            def body(j, _):
                v = buf[pl.ds(j * _L, _L)]
                out_v[pl.ds(j * _L, _L)] = jnp.where(v >= thr_vec, v, fzeros)
                return jnp.int32(0)

            return lax.fori_loop(0, _N // _L, body, jnp.int32(0))

        lax.cond(is_fb, wr_full, wr_scatter)
        out_copy = pltpu.async_copy(out_v, out_hbm.at[base + k], sout)
        if k + 2 < _RPW:
            in_copies[k + 2] = load_row(k + 2)
        prev = (is_fb, pos_v, ncand_c, nv)
    out_copy.wait()


@jax.jit
def kernel(x):
    m, n = x.shape
    run = pl.kernel(
        _sc_body,
        out_type=jax.ShapeDtypeStruct((m, n), jnp.float32),
        mesh=plsc.VectorSubcoreMesh(core_axis_name="c", subcore_axis_name="s"),
        compiler_params=pltpu.CompilerParams(needs_layout_passes=False),
        scratch_types=[
            pltpu.VMEM((_N,), jnp.float32),
            pltpu.VMEM((_N,), jnp.float32),
            pltpu.VMEM((_N,), jnp.float32),
            pltpu.VMEM((_NB,), jnp.int32),
            pltpu.VMEM((_SV * _L,), jnp.int32),
            pltpu.VMEM((_CAP + 2 * _L,), jnp.int32),
            pltpu.VMEM((_CAP + 2 * _L,), jnp.int32),
            pltpu.VMEM((_CAP + 2 * _L,), jnp.int32),
            pltpu.SemaphoreType.DMA((_NCH,)),
            pltpu.SemaphoreType.DMA((_NCH,)),
            pltpu.SemaphoreType.DMA,
        ],
    )
    return run(x)
